# baseline (speedup 1.0000x reference)
import jax
import jax.numpy as jnp
from jax.experimental import pallas as pl
from jax.experimental.pallas import tpu as pltpu

_LANES = 128


def _fused_kernel(na, nb, bm_a, bm,
                  x_ref, adj_ref, w1_ref, b1_ref, w2p_ref, b2p_ref,
                  o_ref, s1_ref, s2_ref):
    t = pl.program_id(0)

    @pl.when(t < na)
    def _phase_a():
        s1_ref[pl.ds(t * bm_a, bm_a), :] = jnp.dot(
            x_ref[...], w1_ref[...], preferred_element_type=jnp.float32)

    @pl.when(jnp.logical_and(t >= na, t < na + nb))
    def _phase_b():
        j = t - na
        acc = jnp.dot(adj_ref[...], s1_ref[...],
                      preferred_element_type=jnp.float32)
        h = jnp.maximum(acc + b1_ref[...], 0.0)
        s2_ref[pl.ds(j * bm, bm), :] = jnp.dot(
            h, w2p_ref[...], preferred_element_type=jnp.float32)

    @pl.when(t >= na + nb)
    def _phase_c():
        acc = jnp.dot(adj_ref[...], s2_ref[:adj_ref.shape[1], :],
                      preferred_element_type=jnp.float32)
        acc = acc + b2p_ref[...]
        m = jnp.max(acc, axis=1, keepdims=True)
        e = jnp.exp(acc - m)
        sm = e / jnp.sum(e, axis=1, keepdims=True)
        o_ref[...] = sm[:, :o_ref.shape[1]]


def _gcn_forward(x, adj, W1, b1, W2, b2, bm_a, bm, interpret=False):
    n, f_in = x.shape
    hid = W1.shape[1]
    f_pad = -(-f_in // _LANES) * _LANES
    if f_pad != f_in:
        x = jnp.pad(x, ((0, 0), (0, f_pad - f_in)))
        W1 = jnp.pad(W1, ((0, f_pad - f_in), (0, 0)))
    f_in = f_pad
    c = W2.shape[1]
    na = n // bm_a
    nb = -(-n // bm)
    n_pad = nb * bm
    b1r = b1.reshape(1, hid)
    w2p = jnp.pad(W2, ((0, 0), (0, _LANES - c)))
    b2p = jnp.pad(b2.reshape(1, c), ((0, 0), (0, _LANES - c)),
                  constant_values=-1e30)

    import functools
    body = functools.partial(_fused_kernel, na, nb, bm_a, bm)

    def x_idx(t):
        return (jnp.minimum(t, na - 1), 0)

    def adj_idx(t):
        return (jnp.where(t < na, 0,
                          jnp.where(t < na + nb, t - na, t - na - nb)), 0)

    def out_idx(t):
        return (jnp.where(t < na + nb, 0, t - na - nb), 0)

    return pl.pallas_call(
        body,
        grid=(na + 2 * nb,),
        in_specs=[
            pl.BlockSpec((bm_a, f_in), x_idx),
            pl.BlockSpec((bm, n), adj_idx),
            pl.BlockSpec((f_in, hid), lambda t: (0, 0)),
            pl.BlockSpec((1, hid), lambda t: (0, 0)),
            pl.BlockSpec((hid, _LANES), lambda t: (0, 0)),
            pl.BlockSpec((1, _LANES), lambda t: (0, 0)),
        ],
        out_specs=pl.BlockSpec((bm, c), out_idx),
        out_shape=jax.ShapeDtypeStruct((n, c), jnp.float32),
        scratch_shapes=[
            pltpu.VMEM((n, hid), jnp.float32),
            pltpu.VMEM((n_pad, _LANES), jnp.float32),
        ],
        interpret=interpret,
    )(x, adj, W1, b1r, w2p, b2p)


def kernel(x, adj, W1, b1, W2, b2):
    return _gcn_forward(x, adj, W1, b1, W2, b2, bm_a=400, bm=320)


# speedup vs baseline: 1.7991x; 1.7991x over previous
import functools

import jax
import jax.numpy as jnp
from jax.experimental import pallas as pl
from jax.experimental.pallas import tpu as pltpu

_LANES = 128


def _fused_kernel(na, nb, bm_a, bm,
                  x_ref, adj_ref, w1_ref, b1_ref, w2p_ref, b2p_ref,
                  o_ref, s1_ref, s2_ref):
    t = pl.program_id(0)

    @pl.when(t < na)
    def _phase_a():
        s1_ref[pl.ds(t * bm_a, bm_a), :] = jnp.dot(
            x_ref[...], w1_ref[...], preferred_element_type=jnp.float32)

    @pl.when(jnp.logical_and(t >= na, t < na + nb))
    def _phase_b():
        j = t - na
        acc = jnp.dot(adj_ref[...], s1_ref[...],
                      preferred_element_type=jnp.float32)
        h = jnp.maximum(acc + b1_ref[...], 0.0)
        s2_ref[pl.ds(j * bm, bm), :] = jnp.dot(
            h, w2p_ref[...], preferred_element_type=jnp.float32)

    @pl.when(t >= na + nb)
    def _phase_c():
        acc = jnp.dot(adj_ref[...], s2_ref[:adj_ref.shape[1], :],
                      preferred_element_type=jnp.float32)
        acc = acc + b2p_ref[...]
        m = jnp.max(acc, axis=1, keepdims=True)
        e = jnp.exp(acc - m)
        sm = e / jnp.sum(e, axis=1, keepdims=True)
        o_ref[...] = sm[:, :o_ref.shape[1]]


def _gcn_forward(x, adj, W1, b1, W2, b2, bm_a, bm, interpret=False):
    n, f_in = x.shape
    hid = W1.shape[1]
    c = W2.shape[1]
    na = n // bm_a
    nb = -(-n // bm)
    n_pad = nb * bm
    b1r = b1.reshape(1, hid)
    w2p = jnp.pad(W2, ((0, 0), (0, _LANES - c)))
    b2p = jnp.pad(b2.reshape(1, c), ((0, 0), (0, _LANES - c)),
                  constant_values=-1e30)

    body = functools.partial(_fused_kernel, na, nb, bm_a, bm)

    def x_idx(t):
        return (jnp.minimum(t, na - 1), 0)

    def adj_idx(t):
        return (jnp.where(t < na, 0,
                          jnp.where(t < na + nb, t - na, t - na - nb)), 0)

    def out_idx(t):
        return (jnp.where(t < na + nb, 0, t - na - nb), 0)

    return pl.pallas_call(
        body,
        grid=(na + 2 * nb,),
        in_specs=[
            pl.BlockSpec((bm_a, f_in), x_idx),
            pl.BlockSpec((bm, n), adj_idx),
            pl.BlockSpec((f_in, hid), lambda t: (0, 0)),
            pl.BlockSpec((1, hid), lambda t: (0, 0)),
            pl.BlockSpec((hid, _LANES), lambda t: (0, 0)),
            pl.BlockSpec((1, _LANES), lambda t: (0, 0)),
        ],
        out_specs=pl.BlockSpec((bm, c), out_idx),
        out_shape=jax.ShapeDtypeStruct((n, c), jnp.float32),
        scratch_shapes=[
            pltpu.VMEM((n, hid), jnp.float32),
            pltpu.VMEM((n_pad, _LANES), jnp.float32),
        ],
        interpret=interpret,
    )(x, adj, W1, b1r, w2p, b2p)


def kernel(x, adj, W1, b1, W2, b2):
    return _gcn_forward(x, adj, W1, b1, W2, b2, bm_a=400, bm=320)


# speedup vs baseline: 1.8419x; 1.0238x over previous
import functools

import jax
import jax.numpy as jnp
from jax.experimental import pallas as pl
from jax.experimental.pallas import tpu as pltpu

_LANES = 128


def _fused_kernel(na, nb, bm_a, bm,
                  x_ref, adj_ref, w1_ref, b1_ref, w2p_ref, b2p_ref,
                  o_ref, s1_ref, s2_ref):
    t = pl.program_id(0)

    @pl.when(t < na)
    def _phase_a():
        s1_ref[pl.ds(t * bm_a, bm_a), :] = jnp.dot(
            x_ref[...], w1_ref[...], preferred_element_type=jnp.float32)

    @pl.when(jnp.logical_and(t >= na, t < na + nb))
    def _phase_b():
        j = t - na
        acc = jnp.dot(adj_ref[...], s1_ref[...],
                      preferred_element_type=jnp.float32)
        h = jnp.maximum(acc + b1_ref[...], 0.0)
        s2_ref[pl.ds(j * bm, bm), :] = jnp.dot(
            h, w2p_ref[...], preferred_element_type=jnp.float32)

    @pl.when(t >= na + nb)
    def _phase_c():
        acc = jnp.dot(adj_ref[...], s2_ref[:adj_ref.shape[1], :],
                      preferred_element_type=jnp.float32)
        acc = acc + b2p_ref[...]
        m = jnp.max(acc, axis=1, keepdims=True)
        e = jnp.exp(acc - m)
        sm = e / jnp.sum(e, axis=1, keepdims=True)
        o_ref[...] = sm[:, :o_ref.shape[1]]


def _gcn_forward(x, adj, W1, b1, W2, b2, bm_a, bm, interpret=False):
    n, f_in = x.shape
    hid = W1.shape[1]
    c = W2.shape[1]
    na = n // bm_a
    nb = -(-n // bm)
    n_pad = nb * bm
    b1r = b1.reshape(1, hid)
    w2p = jnp.pad(W2, ((0, 0), (0, _LANES - c)))
    b2p = jnp.pad(b2.reshape(1, c), ((0, 0), (0, _LANES - c)),
                  constant_values=-1e30)

    body = functools.partial(_fused_kernel, na, nb, bm_a, bm)

    def x_idx(t):
        return (jnp.minimum(t, na - 1), 0)

    def adj_idx(t):
        return (jnp.where(t < na, 0,
                          jnp.where(t < na + nb, t - na, t - na - nb)), 0)

    def out_idx(t):
        return (jnp.where(t < na + nb, 0, t - na - nb), 0)

    return pl.pallas_call(
        body,
        grid=(na + 2 * nb,),
        in_specs=[
            pl.BlockSpec((bm_a, f_in), x_idx),
            pl.BlockSpec((bm, n), adj_idx),
            pl.BlockSpec((f_in, hid), lambda t: (0, 0)),
            pl.BlockSpec((1, hid), lambda t: (0, 0)),
            pl.BlockSpec((hid, _LANES), lambda t: (0, 0)),
            pl.BlockSpec((1, _LANES), lambda t: (0, 0)),
        ],
        out_specs=pl.BlockSpec((bm, c), out_idx),
        out_shape=jax.ShapeDtypeStruct((n, c), jnp.float32),
        scratch_shapes=[
            pltpu.VMEM((n, hid), jnp.float32),
            pltpu.VMEM((n_pad, _LANES), jnp.float32),
        ],
        compiler_params=pltpu.CompilerParams(
            vmem_limit_bytes=67108864),
        interpret=interpret,
    )(x, adj, W1, b1r, w2p, b2p)


def kernel(x, adj, W1, b1, W2, b2):
    return _gcn_forward(x, adj, W1, b1, W2, b2, bm_a=400, bm=400)
